# Initial kernel scaffold; baseline (speedup 1.0000x reference)
import functools

import jax
import jax.numpy as jnp
import numpy as np
from jax import lax
from jax.experimental import pallas as pl
from jax.experimental.pallas import tpu as pltpu
from jax.experimental.pallas import tpu_sc as plsc

_B = 4
_N = 20000
_NPAD = 20480
_ROWS = 160
_SORT_N = 32768
_SORT_ROWS = 256
_G = 100
_POS = 66
_NEG = 134
_TOT = 200
_MH = 28
_MW = 28
_H = 56
_W = 56
_STD = np.asarray([0.1, 0.1, 0.2, 0.2], dtype=np.float32)

_MAXW = np.int32(0x7FFFFFFF)

_CONST = {}


def _selection_ranks():
    if "ranks" not in _CONST:
        base = jax.random.key(42)
        rows = []
        for b in range(_B):
            kb = jax.random.fold_in(base, b)
            perm = jax.random.permutation(kb, _N - _POS)[:_NEG]
            perm = np.asarray(jax.device_get(perm)).astype(np.int64)
            rows.append(np.concatenate([np.arange(_POS), _POS + perm]))
        _CONST["ranks"] = np.stack(rows).astype(np.int32)
    return _CONST["ranks"]


def _iou_key_kernel(gt_ref, p_ref, w_ref):
    b = pl.program_id(0)
    y1 = p_ref[0, 0]
    x1 = p_ref[0, 1]
    y2 = p_ref[0, 2]
    x2 = p_ref[0, 3]
    area1 = (y2 - y1) * (x2 - x1)

    def body(g, m):
        gy1 = gt_ref[b, g, 0]
        gx1 = gt_ref[b, g, 1]
        gy2 = gt_ref[b, g, 2]
        gx2 = gt_ref[b, g, 3]
        iy = jnp.clip(jnp.minimum(y2, gy2) - jnp.maximum(y1, gy1), 0.0)
        ix = jnp.clip(jnp.minimum(x2, gx2) - jnp.maximum(x1, gx1), 0.0)
        inter = iy * ix
        area2 = (gy2 - gy1) * (gx2 - gx1)
        union = area1 + area2 - inter
        return jnp.maximum(m, inter / (union + 1e-8))

    m = lax.fori_loop(0, _G, body, jnp.full((_ROWS, 128), -1.0, jnp.float32))
    u = lax.bitcast_convert_type(m, jnp.int32)
    w_ref[0] = _MAXW - u


def _xor_partner(x, s, axis, bitzero):
    n = x.shape[axis]
    fwd = jnp.concatenate(
        [lax.slice_in_dim(x, s, n, axis=axis), lax.slice_in_dim(x, 0, s, axis=axis)],
        axis=axis)
    bwd = jnp.concatenate(
        [lax.slice_in_dim(x, n - s, n, axis=axis), lax.slice_in_dim(x, 0, n - s, axis=axis)],
        axis=axis)
    return jnp.where(bitzero, fwd, bwd)


def _sort_kernel(ranks_ref, w_ref, p_ref, sel_ref, rois_ref):
    w = jnp.concatenate(
        [w_ref[0], jnp.full((_SORT_ROWS - _ROWS, 128), _MAXW, jnp.int32)], axis=0)
    rr = lax.broadcasted_iota(jnp.int32, (_SORT_ROWS, 128), 0)
    cc = lax.broadcasted_iota(jnp.int32, (_SORT_ROWS, 128), 1)
    idx = rr * 128 + cc

    k = 2
    while k <= _SORT_N:
        j = k // 2
        while j >= 1:
            if j < 128:
                jbit0 = (cc & j) == 0
                ax = 1
                sh = j
            else:
                jbit0 = (rr & (j // 128)) == 0
                ax = 0
                sh = j // 128
            if k < 128:
                kbit0 = (cc & k) == 0
            elif k <= _SORT_N // 2:
                kbit0 = (rr & (k // 128)) == 0
            else:
                kbit0 = jnp.full((_SORT_ROWS, 128), True)
            pw = _xor_partner(w, sh, ax, jbit0)
            pidx = _xor_partner(idx, sh, ax, jbit0)
            keep_small = jnp.logical_not(jnp.logical_xor(jbit0, kbit0))
            le = (w < pw) | ((w == pw) & (idx <= pidx))
            take_self = keep_small == le
            w = jnp.where(take_self, w, pw)
            idx = jnp.where(take_self, idx, pidx)
            j //= 2
        k *= 2

    flatpos = rr * 128 + cc
    ppos = lax.broadcasted_iota(jnp.int32, (_ROWS, 128), 0) * 128 + \
        lax.broadcasted_iota(jnp.int32, (_ROWS, 128), 1)
    py1 = p_ref[0, 0]
    px1 = p_ref[0, 1]
    py2 = p_ref[0, 2]
    px2 = p_ref[0, 3]

    def body(t, _):
        r = ranks_ref[0, t]
        sel = jnp.max(jnp.where(flatpos == r, idx, -1))
        sel_ref[0, t] = sel
        eq = ppos == sel
        rois_ref[0, t, 0] = jnp.max(jnp.where(eq, py1, -1.0))
        rois_ref[0, t, 1] = jnp.max(jnp.where(eq, px1, -1.0))
        rois_ref[0, t, 2] = jnp.max(jnp.where(eq, py2, -1.0))
        rois_ref[0, t, 3] = jnp.max(jnp.where(eq, px2, -1.0))
        return 0

    lax.fori_loop(0, _TOT, body, 0)


def _targets_kernel(r_ref, gt_ref, cls_in_ref, cls_ref, del_ref, rid_ref, flat_ref):
    b = pl.program_id(0)
    y1 = r_ref[0, 0, :_POS]
    x1 = r_ref[0, 1, :_POS]
    y2 = r_ref[0, 2, :_POS]
    x2 = r_ref[0, 3, :_POS]
    gy1 = gt_ref[0, 0]
    gx1 = gt_ref[0, 1]
    gy2 = gt_ref[0, 2]
    gx2 = gt_ref[0, 3]

    iy1 = jnp.maximum(y1[:, None], gy1[None, :])
    ix1 = jnp.maximum(x1[:, None], gx1[None, :])
    iy2 = jnp.minimum(y2[:, None], gy2[None, :])
    ix2 = jnp.minimum(x2[:, None], gx2[None, :])
    inter = jnp.clip(iy2 - iy1, 0.0) * jnp.clip(ix2 - ix1, 0.0)
    area1 = (y2 - y1) * (x2 - x1)
    area2 = (gy2 - gy1) * (gx2 - gx1)
    union = area1[:, None] + area2[None, :] - inter
    ov = inter / (union + 1e-8)
    assign = jnp.argmax(ov, axis=1).astype(jnp.int32)

    eq = assign[:, None] == lax.broadcasted_iota(jnp.int32, (_POS, _G), 1)
    cls_ref[0, 0] = jnp.sum(jnp.where(eq, cls_in_ref[0, 0][None, :], 0), axis=1)
    rid_ref[0, 0] = b * _G + assign

    gby1 = jnp.sum(jnp.where(eq, gy1[None, :], 0.0), axis=1)
    gbx1 = jnp.sum(jnp.where(eq, gx1[None, :], 0.0), axis=1)
    gby2 = jnp.sum(jnp.where(eq, gy2[None, :], 0.0), axis=1)
    gbx2 = jnp.sum(jnp.where(eq, gx2[None, :], 0.0), axis=1)

    h = y2 - y1
    wdt = x2 - x1
    cy = y1 + 0.5 * h
    cx = x1 + 0.5 * wdt
    gh = gby2 - gby1
    gw = gbx2 - gbx1
    gcy = gby1 + 0.5 * gh
    gcx = gbx1 + 0.5 * gw
    del_ref[0, 0] = ((gcy - cy) / h) / _STD[0]
    del_ref[0, 1] = ((gcx - cx) / wdt) / _STD[1]
    del_ref[0, 2] = jnp.log(gh / h) / _STD[2]
    del_ref[0, 3] = jnp.log(gw / wdt) / _STD[3]

    ly = np.linspace(0.0, 1.0, _MH).astype(np.float32)
    lx = np.linspace(0.0, 1.0, _MW).astype(np.float32)
    ys = y1[:, None] + (y2 - y1)[:, None] * ly[None, :]
    xs = x1[:, None] + (x2 - x1)[:, None] * lx[None, :]
    yi = jnp.clip(jnp.round(ys * (_H - 1)), 0, _H - 1).astype(jnp.int32)
    xi = jnp.clip(jnp.round(xs * (_W - 1)), 0, _W - 1).astype(jnp.int32)
    flat_ref[0] = yi[:, :, None] * _W + xi[:, None, :]


_SC_ROWS = 512
_PER_TILE = 16
_MASK_LEN = _H * _W
_SAMP = _MH * _MW


def _sc_mask_gather(masks_t, rowids, flat):
    mesh = plsc.VectorSubcoreMesh(core_axis_name="c", subcore_axis_name="s")

    @functools.partial(
        pl.kernel,
        mesh=mesh,
        out_type=jax.ShapeDtypeStruct((_SC_ROWS, _SAMP), jnp.float32),
        scratch_types=[
            pltpu.VMEM((_PER_TILE,), jnp.int32),
            pltpu.VMEM((_PER_TILE, _MASK_LEN), jnp.float32),
            pltpu.VMEM((_PER_TILE, _SAMP), jnp.int32),
            pltpu.VMEM((_PER_TILE, _SAMP), jnp.float32),
            pltpu.SemaphoreType.DMA,
        ],
    )
    def k(masks_hbm, rowid_hbm, flat_hbm, out_hbm, rid_v, rows_v, flat_v, out_v, sem):
        wid = lax.axis_index("s") * 2 + lax.axis_index("c")
        base = wid * _PER_TILE
        pltpu.sync_copy(rowid_hbm.at[pl.ds(base, _PER_TILE)], rid_v)
        pltpu.async_copy(masks_hbm.at[rid_v], rows_v, sem).wait()
        pltpu.sync_copy(flat_hbm.at[pl.ds(base, _PER_TILE)], flat_v)

        def outer(p, _):
            pidx = jnp.full((16,), p, jnp.int32)

            def inner(c, _):
                iv = flat_v[p, pl.ds(c * 16, 16)]
                vals = plsc.load_gather(rows_v, [pidx, iv])
                out_v[p, pl.ds(c * 16, 16)] = vals
                return 0

            lax.fori_loop(0, _SAMP // 16, inner, 0)
            return 0

        lax.fori_loop(0, _PER_TILE, outer, 0)
        pltpu.sync_copy(out_v, out_hbm.at[pl.ds(base, _PER_TILE)])

    return k(masks_t, rowids, flat)


def kernel(proposals, gt_class_ids, gt_boxes, gt_masks):
    ranks = jnp.asarray(_selection_ranks())

    pt = jnp.transpose(proposals, (0, 2, 1))
    pt = jnp.pad(pt, ((0, 0), (0, 0), (0, _NPAD - _N)))
    pt = pt.reshape(_B, 4, _ROWS, 128)
    gtt = jnp.transpose(gt_boxes, (0, 2, 1))

    w = pl.pallas_call(
        _iou_key_kernel,
        grid=(_B,),
        in_specs=[
            pl.BlockSpec(memory_space=pltpu.SMEM),
            pl.BlockSpec((1, 4, _ROWS, 128), lambda b: (b, 0, 0, 0)),
        ],
        out_specs=pl.BlockSpec((1, _ROWS, 128), lambda b: (b, 0, 0)),
        out_shape=jax.ShapeDtypeStruct((_B, _ROWS, 128), jnp.int32),
    )(gt_boxes, pt)

    sel, rois = pl.pallas_call(
        _sort_kernel,
        grid=(_B,),
        in_specs=[
            pl.BlockSpec((1, _TOT), lambda b: (b, 0), memory_space=pltpu.SMEM),
            pl.BlockSpec((1, _ROWS, 128), lambda b: (b, 0, 0)),
            pl.BlockSpec((1, 4, _ROWS, 128), lambda b: (b, 0, 0, 0)),
        ],
        out_specs=[
            pl.BlockSpec((1, _TOT), lambda b: (b, 0), memory_space=pltpu.SMEM),
            pl.BlockSpec((1, _TOT, 4), lambda b: (b, 0, 0), memory_space=pltpu.SMEM),
        ],
        out_shape=[
            jax.ShapeDtypeStruct((_B, _TOT), jnp.int32),
            jax.ShapeDtypeStruct((_B, _TOT, 4), jnp.float32),
        ],
    )(ranks, w, pt)

    rt = jnp.transpose(rois, (0, 2, 1))
    cls_in = gt_class_ids.reshape(_B, 1, _G)

    cls, dels, rid, flat = pl.pallas_call(
        _targets_kernel,
        grid=(_B,),
        in_specs=[
            pl.BlockSpec((1, 4, _TOT), lambda b: (b, 0, 0)),
            pl.BlockSpec((1, 4, _G), lambda b: (b, 0, 0)),
            pl.BlockSpec((1, 1, _G), lambda b: (b, 0, 0)),
        ],
        out_specs=[
            pl.BlockSpec((1, 1, _POS), lambda b: (b, 0, 0)),
            pl.BlockSpec((1, 4, _POS), lambda b: (b, 0, 0)),
            pl.BlockSpec((1, 1, _POS), lambda b: (b, 0, 0)),
            pl.BlockSpec((1, _POS, _MH, _MW), lambda b: (b, 0, 0, 0)),
        ],
        out_shape=[
            jax.ShapeDtypeStruct((_B, 1, _POS), jnp.int32),
            jax.ShapeDtypeStruct((_B, 4, _POS), jnp.float32),
            jax.ShapeDtypeStruct((_B, 1, _POS), jnp.int32),
            jax.ShapeDtypeStruct((_B, _POS, _MH, _MW), jnp.int32),
        ],
    )(rt, gtt, cls_in)

    masks_t = jnp.transpose(gt_masks, (0, 3, 1, 2)).reshape(_B * _G, _MASK_LEN)
    rid_flat = rid.reshape(_B * _POS)
    pad_ids = (np.arange(_SC_ROWS - _B * _POS) * 7) % (_B * _G)
    rid_pad = jnp.concatenate([rid_flat, jnp.asarray(pad_ids, jnp.int32)])
    flat_pad = jnp.pad(flat.reshape(_B * _POS, _SAMP),
                       ((0, _SC_ROWS - _B * _POS), (0, 0)))

    sampled = _sc_mask_gather(masks_t, rid_pad, flat_pad)

    masks_pos = sampled[: _B * _POS].reshape(_B, _POS, _MH, _MW)
    masks_out = jnp.concatenate(
        [masks_pos, jnp.zeros((_B, _NEG, _MH, _MW), jnp.float32)], axis=1)

    cls_out = jnp.concatenate(
        [cls.reshape(_B, _POS), jnp.zeros((_B, _NEG), jnp.int32)], axis=1)
    dels_out = jnp.concatenate(
        [jnp.transpose(dels, (0, 2, 1)), jnp.zeros((_B, _NEG, 4), jnp.float32)],
        axis=1)
    return rois, cls_out, dels_out, masks_out


# speedup vs baseline: 1.4093x; 1.4093x over previous
import functools

import jax
import jax.numpy as jnp
import numpy as np
from jax import lax
from jax.experimental import pallas as pl
from jax.experimental.pallas import tpu as pltpu
from jax.experimental.pallas import tpu_sc as plsc

_B = 4
_N = 20000
_NPAD = 20480
_ROWS = 160
_SORT_N = 32768
_SORT_ROWS = 256
_G = 100
_POS = 66
_NEG = 134
_TOT = 200
_MH = 28
_MW = 28
_H = 56
_W = 56
_STD = np.asarray([0.1, 0.1, 0.2, 0.2], dtype=np.float32)

_MAXW = np.int32(0x7FFFFFFF)

_CONST = {}


def _selection_ranks():
    if "ranks" not in _CONST:
        base = jax.random.key(42)
        rows = []
        for b in range(_B):
            kb = jax.random.fold_in(base, b)
            perm = jax.random.permutation(kb, _N - _POS)[:_NEG]
            perm = np.asarray(jax.device_get(perm)).astype(np.int64)
            rows.append(np.concatenate([np.arange(_POS), _POS + perm]))
        _CONST["ranks"] = np.stack(rows).astype(np.int32)
    return _CONST["ranks"]


_RANKS = _selection_ranks()


def _iou_key_kernel(gt_ref, p_ref, w_ref):
    b = pl.program_id(0)
    y1 = p_ref[0, 0]
    x1 = p_ref[0, 1]
    y2 = p_ref[0, 2]
    x2 = p_ref[0, 3]
    area1 = (y2 - y1) * (x2 - x1)

    def body(g, m):
        gy1 = gt_ref[b, g, 0]
        gx1 = gt_ref[b, g, 1]
        gy2 = gt_ref[b, g, 2]
        gx2 = gt_ref[b, g, 3]
        iy = jnp.clip(jnp.minimum(y2, gy2) - jnp.maximum(y1, gy1), 0.0)
        ix = jnp.clip(jnp.minimum(x2, gx2) - jnp.maximum(x1, gx1), 0.0)
        inter = iy * ix
        area2 = (gy2 - gy1) * (gx2 - gx1)
        union = area1 + area2 - inter
        return jnp.maximum(m, inter / (union + 1e-8))

    m = lax.fori_loop(0, _G, body, jnp.full((_ROWS, 128), -1.0, jnp.float32))
    u = lax.bitcast_convert_type(m, jnp.int32)
    w_ref[0] = _MAXW - u


def _xor_partner(x, s, axis, bitzero):
    n = x.shape[axis]
    fwd = jnp.concatenate(
        [lax.slice_in_dim(x, s, n, axis=axis), lax.slice_in_dim(x, 0, s, axis=axis)],
        axis=axis)
    bwd = jnp.concatenate(
        [lax.slice_in_dim(x, n - s, n, axis=axis), lax.slice_in_dim(x, 0, n - s, axis=axis)],
        axis=axis)
    return jnp.where(bitzero, fwd, bwd)


def _sort_kernel(ranks_ref, w_ref, p_ref, sel_ref, rois_ref):
    w = jnp.concatenate(
        [w_ref[0], jnp.full((_SORT_ROWS - _ROWS, 128), _MAXW, jnp.int32)], axis=0)
    rr = lax.broadcasted_iota(jnp.int32, (_SORT_ROWS, 128), 0)
    cc = lax.broadcasted_iota(jnp.int32, (_SORT_ROWS, 128), 1)
    idx = rr * 128 + cc

    k = 2
    while k <= _SORT_N:
        j = k // 2
        while j >= 1:
            if j < 128:
                jbit0 = (cc & j) == 0
                ax = 1
                sh = j
            else:
                jbit0 = (rr & (j // 128)) == 0
                ax = 0
                sh = j // 128
            if k < 128:
                kbit0 = (cc & k) == 0
            elif k <= _SORT_N // 2:
                kbit0 = (rr & (k // 128)) == 0
            else:
                kbit0 = jnp.full((_SORT_ROWS, 128), True)
            pw = _xor_partner(w, sh, ax, jbit0)
            pidx = _xor_partner(idx, sh, ax, jbit0)
            keep_small = jnp.logical_not(jnp.logical_xor(jbit0, kbit0))
            le = (w < pw) | ((w == pw) & (idx <= pidx))
            take_self = keep_small == le
            w = jnp.where(take_self, w, pw)
            idx = jnp.where(take_self, idx, pidx)
            j //= 2
        k *= 2

    flatpos = rr * 128 + cc
    ppos = lax.broadcasted_iota(jnp.int32, (_ROWS, 128), 0) * 128 + \
        lax.broadcasted_iota(jnp.int32, (_ROWS, 128), 1)
    py1 = p_ref[0, 0]
    px1 = p_ref[0, 1]
    py2 = p_ref[0, 2]
    px2 = p_ref[0, 3]

    def body(t, _):
        r = ranks_ref[0, 0, t]
        sel = jnp.max(jnp.where(flatpos == r, idx, -1))
        sel_ref[0, 0, t] = sel
        eq = ppos == sel
        rois_ref[0, t, 0] = jnp.max(jnp.where(eq, py1, -1.0))
        rois_ref[0, t, 1] = jnp.max(jnp.where(eq, px1, -1.0))
        rois_ref[0, t, 2] = jnp.max(jnp.where(eq, py2, -1.0))
        rois_ref[0, t, 3] = jnp.max(jnp.where(eq, px2, -1.0))
        return 0

    lax.fori_loop(0, _TOT, body, 0)


def _targets_kernel(r_ref, gt_ref, cls_in_ref, lin_ref, cls_ref, del_ref, rid_ref, flat_ref):
    b = pl.program_id(0)
    y1 = r_ref[0, 0, :_POS]
    x1 = r_ref[0, 1, :_POS]
    y2 = r_ref[0, 2, :_POS]
    x2 = r_ref[0, 3, :_POS]
    gy1 = gt_ref[0, 0]
    gx1 = gt_ref[0, 1]
    gy2 = gt_ref[0, 2]
    gx2 = gt_ref[0, 3]

    iy1 = jnp.maximum(y1[:, None], gy1[None, :])
    ix1 = jnp.maximum(x1[:, None], gx1[None, :])
    iy2 = jnp.minimum(y2[:, None], gy2[None, :])
    ix2 = jnp.minimum(x2[:, None], gx2[None, :])
    inter = jnp.clip(iy2 - iy1, 0.0) * jnp.clip(ix2 - ix1, 0.0)
    area1 = (y2 - y1) * (x2 - x1)
    area2 = (gy2 - gy1) * (gx2 - gx1)
    union = area1[:, None] + area2[None, :] - inter
    ov = inter / (union + 1e-8)
    assign = jnp.argmax(ov, axis=1).astype(jnp.int32)

    eq = assign[:, None] == lax.broadcasted_iota(jnp.int32, (_POS, _G), 1)
    cls_ref[0, 0] = jnp.sum(jnp.where(eq, cls_in_ref[0, 0][None, :], 0), axis=1)
    rid_ref[0, 0] = b * _G + assign

    gby1 = jnp.sum(jnp.where(eq, gy1[None, :], 0.0), axis=1)
    gbx1 = jnp.sum(jnp.where(eq, gx1[None, :], 0.0), axis=1)
    gby2 = jnp.sum(jnp.where(eq, gy2[None, :], 0.0), axis=1)
    gbx2 = jnp.sum(jnp.where(eq, gx2[None, :], 0.0), axis=1)

    h = y2 - y1
    wdt = x2 - x1
    cy = y1 + 0.5 * h
    cx = x1 + 0.5 * wdt
    gh = gby2 - gby1
    gw = gbx2 - gbx1
    gcy = gby1 + 0.5 * gh
    gcx = gbx1 + 0.5 * gw
    del_ref[0, 0] = ((gcy - cy) / h) / _STD[0]
    del_ref[0, 1] = ((gcx - cx) / wdt) / _STD[1]
    del_ref[0, 2] = jnp.log(gh / h) / _STD[2]
    del_ref[0, 3] = jnp.log(gw / wdt) / _STD[3]

    ly = lin_ref[0]
    lx = lin_ref[1]
    ys = y1[:, None] + (y2 - y1)[:, None] * ly[None, :]
    xs = x1[:, None] + (x2 - x1)[:, None] * lx[None, :]
    yi = jnp.clip(jnp.round(ys * (_H - 1)), 0, _H - 1).astype(jnp.int32)
    xi = jnp.clip(jnp.round(xs * (_W - 1)), 0, _W - 1).astype(jnp.int32)
    flat_ref[0] = yi[:, :, None] * _W + xi[:, None, :]


_SC_ROWS = 512
_PER_TILE = 16
_MASK_LEN = 3200
_SAMP = _MH * _MW


def _sc_mask_gather(masks_t, rowids, flat):
    mesh = plsc.VectorSubcoreMesh(core_axis_name="c", subcore_axis_name="s")

    @functools.partial(
        pl.kernel,
        mesh=mesh,
        compiler_params=pltpu.CompilerParams(needs_layout_passes=False),
        out_type=jax.ShapeDtypeStruct((_SC_ROWS, _SAMP), jnp.float32),
        scratch_types=[
            pltpu.VMEM((_PER_TILE,), jnp.int32),
            pltpu.VMEM((_PER_TILE, _MASK_LEN), jnp.float32),
            pltpu.VMEM((_PER_TILE, _SAMP), jnp.int32),
            pltpu.VMEM((_PER_TILE, _SAMP), jnp.float32),
            pltpu.SemaphoreType.DMA,
        ],
    )
    def k(masks_hbm, rowid_hbm, flat_hbm, out_hbm, rid_v, rows_v, flat_v, out_v, sem):
        wid = lax.axis_index("s") * 2 + lax.axis_index("c")
        base = wid * _PER_TILE
        pltpu.sync_copy(rowid_hbm.at[pl.ds(base, _PER_TILE)], rid_v)
        pltpu.async_copy(masks_hbm.at[rid_v], rows_v, sem).wait()
        pltpu.sync_copy(flat_hbm.at[pl.ds(base, _PER_TILE)], flat_v)

        def outer(p, _):
            pidx = jnp.full((16,), p, jnp.int32)

            def inner(c, _):
                iv = flat_v[p, pl.ds(c * 16, 16)]
                vals = plsc.load_gather(rows_v, [pidx, iv])
                out_v[p, pl.ds(c * 16, 16)] = vals
                return 0

            lax.fori_loop(0, _SAMP // 16, inner, 0)
            return 0

        lax.fori_loop(0, _PER_TILE, outer, 0)
        pltpu.sync_copy(out_v, out_hbm.at[pl.ds(base, _PER_TILE)])

    return k(masks_t, rowids, flat)


def kernel(proposals, gt_class_ids, gt_boxes, gt_masks):
    ranks = jnp.asarray(_RANKS)

    pt = jnp.transpose(proposals, (0, 2, 1))
    pt = jnp.pad(pt, ((0, 0), (0, 0), (0, _NPAD - _N)))
    pt = pt.reshape(_B, 4, _ROWS, 128)
    gtt = jnp.transpose(gt_boxes, (0, 2, 1))

    w = pl.pallas_call(
        _iou_key_kernel,
        grid=(_B,),
        in_specs=[
            pl.BlockSpec(memory_space=pltpu.SMEM),
            pl.BlockSpec((1, 4, _ROWS, 128), lambda b: (b, 0, 0, 0)),
        ],
        out_specs=pl.BlockSpec((1, _ROWS, 128), lambda b: (b, 0, 0)),
        out_shape=jax.ShapeDtypeStruct((_B, _ROWS, 128), jnp.int32),
    )(gt_boxes, pt)

    sel, rois = pl.pallas_call(
        _sort_kernel,
        grid=(_B,),
        in_specs=[
            pl.BlockSpec((1, 1, _TOT), lambda b: (b, 0, 0), memory_space=pltpu.SMEM),
            pl.BlockSpec((1, _ROWS, 128), lambda b: (b, 0, 0)),
            pl.BlockSpec((1, 4, _ROWS, 128), lambda b: (b, 0, 0, 0)),
        ],
        out_specs=[
            pl.BlockSpec((1, 1, _TOT), lambda b: (b, 0, 0), memory_space=pltpu.SMEM),
            pl.BlockSpec((1, _TOT, 4), lambda b: (b, 0, 0), memory_space=pltpu.SMEM),
        ],
        out_shape=[
            jax.ShapeDtypeStruct((_B, 1, _TOT), jnp.int32),
            jax.ShapeDtypeStruct((_B, _TOT, 4), jnp.float32),
        ],
    )(ranks.reshape(_B, 1, _TOT), w, pt)

    rt = jnp.transpose(rois, (0, 2, 1))
    cls_in = gt_class_ids.reshape(_B, 1, _G)
    lin = jnp.stack([jnp.linspace(0.0, 1.0, _MH), jnp.linspace(0.0, 1.0, _MW)])

    cls, dels, rid, flat = pl.pallas_call(
        _targets_kernel,
        grid=(_B,),
        in_specs=[
            pl.BlockSpec((1, 4, _TOT), lambda b: (b, 0, 0)),
            pl.BlockSpec((1, 4, _G), lambda b: (b, 0, 0)),
            pl.BlockSpec((1, 1, _G), lambda b: (b, 0, 0)),
            pl.BlockSpec((2, _MH), lambda b: (0, 0)),
        ],
        out_specs=[
            pl.BlockSpec((1, 1, _POS), lambda b: (b, 0, 0)),
            pl.BlockSpec((1, 4, _POS), lambda b: (b, 0, 0)),
            pl.BlockSpec((1, 1, _POS), lambda b: (b, 0, 0)),
            pl.BlockSpec((1, _POS, _MH, _MW), lambda b: (b, 0, 0, 0)),
        ],
        out_shape=[
            jax.ShapeDtypeStruct((_B, 1, _POS), jnp.int32),
            jax.ShapeDtypeStruct((_B, 4, _POS), jnp.float32),
            jax.ShapeDtypeStruct((_B, 1, _POS), jnp.int32),
            jax.ShapeDtypeStruct((_B, _POS, _MH, _MW), jnp.int32),
        ],
    )(rt, gtt, cls_in, lin.astype(jnp.float32))

    masks_t = jnp.transpose(gt_masks, (0, 3, 1, 2)).reshape(_B * _G, _H * _W)
    masks_t = jnp.pad(masks_t, ((0, 0), (0, _MASK_LEN - _H * _W)))
    rid_flat = rid.reshape(_B * _POS)
    pad_ids = (np.arange(_SC_ROWS - _B * _POS) * 7) % (_B * _G)
    rid_pad = jnp.concatenate([rid_flat, jnp.asarray(pad_ids, jnp.int32)])
    flat_pad = jnp.pad(flat.reshape(_B * _POS, _SAMP),
                       ((0, _SC_ROWS - _B * _POS), (0, 0)))

    sampled = _sc_mask_gather(masks_t, rid_pad, flat_pad)

    masks_pos = sampled[: _B * _POS].reshape(_B, _POS, _MH, _MW)
    masks_out = jnp.concatenate(
        [masks_pos, jnp.zeros((_B, _NEG, _MH, _MW), jnp.float32)], axis=1)

    cls_out = jnp.concatenate(
        [cls.reshape(_B, _POS), jnp.zeros((_B, _NEG), jnp.int32)], axis=1)
    dels_out = jnp.concatenate(
        [jnp.transpose(dels, (0, 2, 1)), jnp.zeros((_B, _NEG, 4), jnp.float32)],
        axis=1)
    return rois, cls_out, dels_out, masks_out


# speedup vs baseline: 1.4155x; 1.0044x over previous
import functools

import jax
import jax.numpy as jnp
import numpy as np
from jax import lax
from jax.experimental import pallas as pl
from jax.experimental.pallas import tpu as pltpu
from jax.experimental.pallas import tpu_sc as plsc

_B = 4
_N = 20000
_NPAD = 20480
_ROWS = 160
_SORT_N = 32768
_SORT_ROWS = 256
_G = 100
_POS = 66
_NEG = 134
_TOT = 200
_MH = 28
_MW = 28
_H = 56
_W = 56
_STD = np.asarray([0.1, 0.1, 0.2, 0.2], dtype=np.float32)

_MAXW = np.int32(0x7FFFFFFF)

_CONST = {}


def _selection_ranks():
    if "ranks" not in _CONST:
        base = jax.random.key(42)
        rows = []
        for b in range(_B):
            kb = jax.random.fold_in(base, b)
            perm = jax.random.permutation(kb, _N - _POS)[:_NEG]
            perm = np.asarray(jax.device_get(perm)).astype(np.int64)
            rows.append(np.concatenate([np.arange(_POS), _POS + perm]))
        _CONST["ranks"] = np.stack(rows).astype(np.int32)
    return _CONST["ranks"]


_RANKS = _selection_ranks()


def _xor_partner(x, s, axis, bitzero):
    n = x.shape[axis]
    fwd = pltpu.roll(x, n - s, axis)
    bwd = pltpu.roll(x, s, axis)
    return jnp.where(bitzero, fwd, bwd)


def _sort_kernel(ranks_ref, gt_ref, p_ref, sel_ref, rois_ref):
    b = pl.program_id(0)
    y1 = p_ref[0, 0]
    x1 = p_ref[0, 1]
    y2 = p_ref[0, 2]
    x2 = p_ref[0, 3]
    area1 = (y2 - y1) * (x2 - x1)

    def iou_body(g, m):
        gy1 = gt_ref[b, g, 0]
        gx1 = gt_ref[b, g, 1]
        gy2 = gt_ref[b, g, 2]
        gx2 = gt_ref[b, g, 3]
        iy = jnp.clip(jnp.minimum(y2, gy2) - jnp.maximum(y1, gy1), 0.0)
        ix = jnp.clip(jnp.minimum(x2, gx2) - jnp.maximum(x1, gx1), 0.0)
        inter = iy * ix
        area2 = (gy2 - gy1) * (gx2 - gx1)
        union = area1 + area2 - inter
        return jnp.maximum(m, inter / (union + 1e-8))

    m = lax.fori_loop(0, _G, iou_body,
                      jnp.full((_ROWS, 128), -1.0, jnp.float32))
    wkey = _MAXW - lax.bitcast_convert_type(m, jnp.int32)

    w = jnp.concatenate(
        [wkey, jnp.full((_SORT_ROWS - _ROWS, 128), _MAXW, jnp.int32)], axis=0)
    rr = lax.broadcasted_iota(jnp.int32, (_SORT_ROWS, 128), 0)
    cc = lax.broadcasted_iota(jnp.int32, (_SORT_ROWS, 128), 1)
    idx = rr * 128 + cc

    k = 2
    while k <= _SORT_N:
        j = k // 2
        while j >= 1:
            if j < 128:
                jbit0 = (cc & j) == 0
                ax = 1
                sh = j
            else:
                jbit0 = (rr & (j // 128)) == 0
                ax = 0
                sh = j // 128
            if k < 128:
                kbit0 = (cc & k) == 0
            elif k <= _SORT_N // 2:
                kbit0 = (rr & (k // 128)) == 0
            else:
                kbit0 = jnp.full((_SORT_ROWS, 128), True)
            pw = _xor_partner(w, sh, ax, jbit0)
            pidx = _xor_partner(idx, sh, ax, jbit0)
            keep_small = jnp.logical_not(jnp.logical_xor(jbit0, kbit0))
            le = (w < pw) | ((w == pw) & (idx <= pidx))
            take_self = keep_small == le
            w = jnp.where(take_self, w, pw)
            idx = jnp.where(take_self, idx, pidx)
            j //= 2
        k *= 2

    flatpos = rr * 128 + cc
    ppos = lax.broadcasted_iota(jnp.int32, (_ROWS, 128), 0) * 128 + \
        lax.broadcasted_iota(jnp.int32, (_ROWS, 128), 1)
    py1 = p_ref[0, 0]
    px1 = p_ref[0, 1]
    py2 = p_ref[0, 2]
    px2 = p_ref[0, 3]

    def body(t, _):
        r = ranks_ref[0, 0, t]
        sel = jnp.max(jnp.where(flatpos == r, idx, -1))
        sel_ref[0, 0, t] = sel
        eq = ppos == sel
        rois_ref[0, t, 0] = jnp.max(jnp.where(eq, py1, -1.0))
        rois_ref[0, t, 1] = jnp.max(jnp.where(eq, px1, -1.0))
        rois_ref[0, t, 2] = jnp.max(jnp.where(eq, py2, -1.0))
        rois_ref[0, t, 3] = jnp.max(jnp.where(eq, px2, -1.0))
        return 0

    lax.fori_loop(0, _TOT, body, 0)


def _targets_kernel(r_ref, gt_ref, cls_in_ref, lin_ref, cls_ref, del_ref, rid_ref, flat_ref):
    b = pl.program_id(0)
    y1 = r_ref[0, 0, :_POS]
    x1 = r_ref[0, 1, :_POS]
    y2 = r_ref[0, 2, :_POS]
    x2 = r_ref[0, 3, :_POS]
    gy1 = gt_ref[0, 0]
    gx1 = gt_ref[0, 1]
    gy2 = gt_ref[0, 2]
    gx2 = gt_ref[0, 3]

    iy1 = jnp.maximum(y1[:, None], gy1[None, :])
    ix1 = jnp.maximum(x1[:, None], gx1[None, :])
    iy2 = jnp.minimum(y2[:, None], gy2[None, :])
    ix2 = jnp.minimum(x2[:, None], gx2[None, :])
    inter = jnp.clip(iy2 - iy1, 0.0) * jnp.clip(ix2 - ix1, 0.0)
    area1 = (y2 - y1) * (x2 - x1)
    area2 = (gy2 - gy1) * (gx2 - gx1)
    union = area1[:, None] + area2[None, :] - inter
    ov = inter / (union + 1e-8)
    assign = jnp.argmax(ov, axis=1).astype(jnp.int32)

    eq = assign[:, None] == lax.broadcasted_iota(jnp.int32, (_POS, _G), 1)
    cls_ref[0, 0] = jnp.sum(jnp.where(eq, cls_in_ref[0, 0][None, :], 0), axis=1)
    rid_ref[0, 0] = b * _G + assign

    gby1 = jnp.sum(jnp.where(eq, gy1[None, :], 0.0), axis=1)
    gbx1 = jnp.sum(jnp.where(eq, gx1[None, :], 0.0), axis=1)
    gby2 = jnp.sum(jnp.where(eq, gy2[None, :], 0.0), axis=1)
    gbx2 = jnp.sum(jnp.where(eq, gx2[None, :], 0.0), axis=1)

    h = y2 - y1
    wdt = x2 - x1
    cy = y1 + 0.5 * h
    cx = x1 + 0.5 * wdt
    gh = gby2 - gby1
    gw = gbx2 - gbx1
    gcy = gby1 + 0.5 * gh
    gcx = gbx1 + 0.5 * gw
    del_ref[0, 0] = ((gcy - cy) / h) / _STD[0]
    del_ref[0, 1] = ((gcx - cx) / wdt) / _STD[1]
    del_ref[0, 2] = jnp.log(gh / h) / _STD[2]
    del_ref[0, 3] = jnp.log(gw / wdt) / _STD[3]

    ly = lin_ref[0]
    lx = lin_ref[1]
    ys = y1[:, None] + (y2 - y1)[:, None] * ly[None, :]
    xs = x1[:, None] + (x2 - x1)[:, None] * lx[None, :]
    yi = jnp.clip(jnp.round(ys * (_H - 1)), 0, _H - 1).astype(jnp.int32)
    xi = jnp.clip(jnp.round(xs * (_W - 1)), 0, _W - 1).astype(jnp.int32)
    flat_ref[0] = yi[:, :, None] * _W + xi[:, None, :]


_SC_ROWS = 512
_PER_TILE = 16
_MASK_LEN = 3200
_SAMP = _MH * _MW


def _sc_mask_gather(masks_t, rowids, flat):
    mesh = plsc.VectorSubcoreMesh(core_axis_name="c", subcore_axis_name="s")

    @functools.partial(
        pl.kernel,
        mesh=mesh,
        compiler_params=pltpu.CompilerParams(needs_layout_passes=False),
        out_type=jax.ShapeDtypeStruct((_SC_ROWS, _SAMP), jnp.float32),
        scratch_types=[
            pltpu.VMEM((_PER_TILE,), jnp.int32),
            pltpu.VMEM((_PER_TILE, _MASK_LEN), jnp.float32),
            pltpu.VMEM((_PER_TILE, _SAMP), jnp.int32),
            pltpu.VMEM((_PER_TILE, _SAMP), jnp.float32),
            pltpu.SemaphoreType.DMA,
        ],
    )
    def k(masks_hbm, rowid_hbm, flat_hbm, out_hbm, rid_v, rows_v, flat_v, out_v, sem):
        wid = lax.axis_index("s") * 2 + lax.axis_index("c")
        base = wid * _PER_TILE
        pltpu.sync_copy(rowid_hbm.at[pl.ds(base, _PER_TILE)], rid_v)
        pltpu.async_copy(masks_hbm.at[rid_v], rows_v, sem).wait()
        pltpu.sync_copy(flat_hbm.at[pl.ds(base, _PER_TILE)], flat_v)

        def outer(p, _):
            pidx = jnp.full((16,), p, jnp.int32)

            def inner(c, _):
                iv = flat_v[p, pl.ds(c * 16, 16)]
                vals = plsc.load_gather(rows_v, [pidx, iv])
                out_v[p, pl.ds(c * 16, 16)] = vals
                return 0

            lax.fori_loop(0, _SAMP // 16, inner, 0)
            return 0

        lax.fori_loop(0, _PER_TILE, outer, 0)
        pltpu.sync_copy(out_v, out_hbm.at[pl.ds(base, _PER_TILE)])

    return k(masks_t, rowids, flat)


def kernel(proposals, gt_class_ids, gt_boxes, gt_masks):
    ranks = jnp.asarray(_RANKS)

    pt = jnp.transpose(proposals, (0, 2, 1))
    pt = jnp.pad(pt, ((0, 0), (0, 0), (0, _NPAD - _N)))
    pt = pt.reshape(_B, 4, _ROWS, 128)
    gtt = jnp.transpose(gt_boxes, (0, 2, 1))

    sel, rois = pl.pallas_call(
        _sort_kernel,
        grid=(_B,),
        in_specs=[
            pl.BlockSpec((1, 1, _TOT), lambda b: (b, 0, 0), memory_space=pltpu.SMEM),
            pl.BlockSpec(memory_space=pltpu.SMEM),
            pl.BlockSpec((1, 4, _ROWS, 128), lambda b: (b, 0, 0, 0)),
        ],
        out_specs=[
            pl.BlockSpec((1, 1, _TOT), lambda b: (b, 0, 0), memory_space=pltpu.SMEM),
            pl.BlockSpec((1, _TOT, 4), lambda b: (b, 0, 0), memory_space=pltpu.SMEM),
        ],
        out_shape=[
            jax.ShapeDtypeStruct((_B, 1, _TOT), jnp.int32),
            jax.ShapeDtypeStruct((_B, _TOT, 4), jnp.float32),
        ],
    )(ranks.reshape(_B, 1, _TOT), gt_boxes, pt)

    rt = jnp.transpose(rois, (0, 2, 1))
    cls_in = gt_class_ids.reshape(_B, 1, _G)
    lin = jnp.stack([jnp.linspace(0.0, 1.0, _MH), jnp.linspace(0.0, 1.0, _MW)])

    cls, dels, rid, flat = pl.pallas_call(
        _targets_kernel,
        grid=(_B,),
        in_specs=[
            pl.BlockSpec((1, 4, _TOT), lambda b: (b, 0, 0)),
            pl.BlockSpec((1, 4, _G), lambda b: (b, 0, 0)),
            pl.BlockSpec((1, 1, _G), lambda b: (b, 0, 0)),
            pl.BlockSpec((2, _MH), lambda b: (0, 0)),
        ],
        out_specs=[
            pl.BlockSpec((1, 1, _POS), lambda b: (b, 0, 0)),
            pl.BlockSpec((1, 4, _POS), lambda b: (b, 0, 0)),
            pl.BlockSpec((1, 1, _POS), lambda b: (b, 0, 0)),
            pl.BlockSpec((1, _POS, _MH, _MW), lambda b: (b, 0, 0, 0)),
        ],
        out_shape=[
            jax.ShapeDtypeStruct((_B, 1, _POS), jnp.int32),
            jax.ShapeDtypeStruct((_B, 4, _POS), jnp.float32),
            jax.ShapeDtypeStruct((_B, 1, _POS), jnp.int32),
            jax.ShapeDtypeStruct((_B, _POS, _MH, _MW), jnp.int32),
        ],
    )(rt, gtt, cls_in, lin.astype(jnp.float32))

    masks_t = jnp.transpose(gt_masks, (0, 3, 1, 2)).reshape(_B * _G, _H * _W)
    masks_t = jnp.pad(masks_t, ((0, 0), (0, _MASK_LEN - _H * _W)))
    rid_flat = rid.reshape(_B * _POS)
    pad_ids = (np.arange(_SC_ROWS - _B * _POS) * 7) % (_B * _G)
    rid_pad = jnp.concatenate([rid_flat, jnp.asarray(pad_ids, jnp.int32)])
    flat_pad = jnp.pad(flat.reshape(_B * _POS, _SAMP),
                       ((0, _SC_ROWS - _B * _POS), (0, 0)))

    sampled = _sc_mask_gather(masks_t, rid_pad, flat_pad)

    masks_pos = sampled[: _B * _POS].reshape(_B, _POS, _MH, _MW)
    masks_out = jnp.concatenate(
        [masks_pos, jnp.zeros((_B, _NEG, _MH, _MW), jnp.float32)], axis=1)

    cls_out = jnp.concatenate(
        [cls.reshape(_B, _POS), jnp.zeros((_B, _NEG), jnp.int32)], axis=1)
    dels_out = jnp.concatenate(
        [jnp.transpose(dels, (0, 2, 1)), jnp.zeros((_B, _NEG, 4), jnp.float32)],
        axis=1)
    return rois, cls_out, dels_out, masks_out


# speedup vs baseline: 2.5531x; 1.8037x over previous
import functools

import jax
import jax.numpy as jnp
import numpy as np
from jax import lax
from jax.experimental import pallas as pl
from jax.experimental.pallas import tpu as pltpu
from jax.experimental.pallas import tpu_sc as plsc

_B = 4
_N = 20000
_NPAD = 20480
_ROWS = 160
_SORT_N = 32768
_SORT_ROWS = 256
_G = 100
_POS = 66
_NEG = 134
_TOT = 200
_MH = 28
_MW = 28
_H = 56
_W = 56
_STD = np.asarray([0.1, 0.1, 0.2, 0.2], dtype=np.float32)

_MAXW = np.int32(0x7FFFFFFF)

_CONST = {}


def _selection_ranks():
    if "ranks" not in _CONST:
        base = jax.random.key(42)
        rows = []
        for b in range(_B):
            kb = jax.random.fold_in(base, b)
            perm = jax.random.permutation(kb, _N - _POS)[:_NEG]
            perm = np.asarray(jax.device_get(perm)).astype(np.int64)
            rows.append(np.concatenate([np.arange(_POS), _POS + perm]))
        _CONST["ranks"] = np.stack(rows).astype(np.int32)
    return _CONST["ranks"]


_RANKS = _selection_ranks()

_RANKS32 = np.zeros((32, 32), np.int32)
for _b in range(_B):
    for _s in range(8):
        _RANKS32[_b * 8 + _s, :25] = _RANKS[_b, _s * 25:(_s + 1) * 25]
_SEL_COLS = np.asarray([(t // 25) * 32 + t % 25 for t in range(_TOT)], np.int32)


def _xor_partner(x, s, axis, bitzero):
    n = x.shape[axis]
    fwd = pltpu.roll(x, n - s, axis)
    bwd = pltpu.roll(x, s, axis)
    return jnp.where(bitzero, fwd, bwd)


def _sort_kernel(gt_ref, p_ref, order_ref):
    b = pl.program_id(0)
    y1 = p_ref[0, 0]
    x1 = p_ref[0, 1]
    y2 = p_ref[0, 2]
    x2 = p_ref[0, 3]
    area1 = (y2 - y1) * (x2 - x1)

    def iou_body(g, m):
        gy1 = gt_ref[b, g, 0]
        gx1 = gt_ref[b, g, 1]
        gy2 = gt_ref[b, g, 2]
        gx2 = gt_ref[b, g, 3]
        iy = jnp.clip(jnp.minimum(y2, gy2) - jnp.maximum(y1, gy1), 0.0)
        ix = jnp.clip(jnp.minimum(x2, gx2) - jnp.maximum(x1, gx1), 0.0)
        inter = iy * ix
        area2 = (gy2 - gy1) * (gx2 - gx1)
        union = area1 + area2 - inter
        return jnp.maximum(m, inter / (union + 1e-8))

    m = lax.fori_loop(0, _G, iou_body,
                      jnp.full((_ROWS, 128), -1.0, jnp.float32))
    wkey = _MAXW - lax.bitcast_convert_type(m, jnp.int32)

    w = jnp.concatenate(
        [wkey, jnp.full((_SORT_ROWS - _ROWS, 128), _MAXW, jnp.int32)], axis=0)
    rr = lax.broadcasted_iota(jnp.int32, (_SORT_ROWS, 128), 0)
    cc = lax.broadcasted_iota(jnp.int32, (_SORT_ROWS, 128), 1)
    idx = rr * 128 + cc

    k = 2
    while k <= _SORT_N:
        j = k // 2
        while j >= 1:
            if j < 128:
                jbit0 = (cc & j) == 0
                ax = 1
                sh = j
            else:
                jbit0 = (rr & (j // 128)) == 0
                ax = 0
                sh = j // 128
            if k < 128:
                kbit0 = (cc & k) == 0
            elif k <= _SORT_N // 2:
                kbit0 = (rr & (k // 128)) == 0
            else:
                kbit0 = jnp.full((_SORT_ROWS, 128), True)
            pw = _xor_partner(w, sh, ax, jbit0)
            pidx = _xor_partner(idx, sh, ax, jbit0)
            keep_small = jnp.logical_not(jnp.logical_xor(jbit0, kbit0))
            le = (w < pw) | ((w == pw) & (idx <= pidx))
            take_self = keep_small == le
            w = jnp.where(take_self, w, pw)
            idx = jnp.where(take_self, idx, pidx)
            j //= 2
        k *= 2

    order_ref[0] = idx


_SEL_PER_TILE = 32


def _sc_select(sorted_idx, props, ranks32):
    mesh = plsc.VectorSubcoreMesh(core_axis_name="c", subcore_axis_name="s")

    @functools.partial(
        pl.kernel,
        mesh=mesh,
        compiler_params=pltpu.CompilerParams(needs_layout_passes=False),
        out_type=jax.ShapeDtypeStruct((32, 4, _SEL_PER_TILE), jnp.float32),
        scratch_types=[
            pltpu.VMEM((_SORT_N,), jnp.int32),
            pltpu.VMEM((_N * 4,), jnp.float32),
            pltpu.VMEM((_SEL_PER_TILE,), jnp.int32),
            pltpu.VMEM((4, _SEL_PER_TILE), jnp.float32),
        ],
    )
    def k(sorted_hbm, props_hbm, ranks_hbm, out_hbm, sort_v, props_v, ranks_v, out_v):
        wid = lax.axis_index("s") * 2 + lax.axis_index("c")
        img = wid // 8
        pltpu.sync_copy(sorted_hbm.at[img], sort_v)
        pltpu.sync_copy(props_hbm.at[img], props_v)
        pltpu.sync_copy(ranks_hbm.at[wid], ranks_v)

        def body(c, _):
            ri = ranks_v[pl.ds(c * 16, 16)]
            sel16 = plsc.load_gather(sort_v, [ri])
            for c4 in range(4):
                vals = plsc.load_gather(props_v, [sel16 * 4 + c4])
                out_v[c4, pl.ds(c * 16, 16)] = vals
            return 0

        lax.fori_loop(0, _SEL_PER_TILE // 16, body, 0)
        pltpu.sync_copy(out_v, out_hbm.at[wid])

    return k(sorted_idx, props, ranks32)


def _targets_kernel(r_ref, gt_ref, cls_in_ref, lin_ref, cls_ref, del_ref, rid_ref, flat_ref):
    b = pl.program_id(0)
    y1 = r_ref[0, 0, :_POS]
    x1 = r_ref[0, 1, :_POS]
    y2 = r_ref[0, 2, :_POS]
    x2 = r_ref[0, 3, :_POS]
    gy1 = gt_ref[0, 0]
    gx1 = gt_ref[0, 1]
    gy2 = gt_ref[0, 2]
    gx2 = gt_ref[0, 3]

    iy1 = jnp.maximum(y1[:, None], gy1[None, :])
    ix1 = jnp.maximum(x1[:, None], gx1[None, :])
    iy2 = jnp.minimum(y2[:, None], gy2[None, :])
    ix2 = jnp.minimum(x2[:, None], gx2[None, :])
    inter = jnp.clip(iy2 - iy1, 0.0) * jnp.clip(ix2 - ix1, 0.0)
    area1 = (y2 - y1) * (x2 - x1)
    area2 = (gy2 - gy1) * (gx2 - gx1)
    union = area1[:, None] + area2[None, :] - inter
    ov = inter / (union + 1e-8)
    assign = jnp.argmax(ov, axis=1).astype(jnp.int32)

    eq = assign[:, None] == lax.broadcasted_iota(jnp.int32, (_POS, _G), 1)
    cls_ref[0, 0] = jnp.sum(jnp.where(eq, cls_in_ref[0, 0][None, :], 0), axis=1)
    rid_ref[0, 0] = b * _G + assign

    gby1 = jnp.sum(jnp.where(eq, gy1[None, :], 0.0), axis=1)
    gbx1 = jnp.sum(jnp.where(eq, gx1[None, :], 0.0), axis=1)
    gby2 = jnp.sum(jnp.where(eq, gy2[None, :], 0.0), axis=1)
    gbx2 = jnp.sum(jnp.where(eq, gx2[None, :], 0.0), axis=1)

    h = y2 - y1
    wdt = x2 - x1
    cy = y1 + 0.5 * h
    cx = x1 + 0.5 * wdt
    gh = gby2 - gby1
    gw = gbx2 - gbx1
    gcy = gby1 + 0.5 * gh
    gcx = gbx1 + 0.5 * gw
    del_ref[0, 0] = ((gcy - cy) / h) / _STD[0]
    del_ref[0, 1] = ((gcx - cx) / wdt) / _STD[1]
    del_ref[0, 2] = jnp.log(gh / h) / _STD[2]
    del_ref[0, 3] = jnp.log(gw / wdt) / _STD[3]

    ly = lin_ref[0]
    lx = lin_ref[1]
    ys = y1[:, None] + (y2 - y1)[:, None] * ly[None, :]
    xs = x1[:, None] + (x2 - x1)[:, None] * lx[None, :]
    yi = jnp.clip(jnp.round(ys * (_H - 1)), 0, _H - 1).astype(jnp.int32)
    xi = jnp.clip(jnp.round(xs * (_W - 1)), 0, _W - 1).astype(jnp.int32)
    flat_ref[0] = yi[:, :, None] * _W + xi[:, None, :]


_SC_ROWS = 512
_PER_TILE = 16
_MASK_LEN = 3200
_SAMP = _MH * _MW


def _sc_mask_gather(masks_t, rowids, flat):
    mesh = plsc.VectorSubcoreMesh(core_axis_name="c", subcore_axis_name="s")

    @functools.partial(
        pl.kernel,
        mesh=mesh,
        compiler_params=pltpu.CompilerParams(needs_layout_passes=False),
        out_type=jax.ShapeDtypeStruct((_SC_ROWS, _SAMP), jnp.float32),
        scratch_types=[
            pltpu.VMEM((_PER_TILE,), jnp.int32),
            pltpu.VMEM((_PER_TILE, _MASK_LEN), jnp.float32),
            pltpu.VMEM((_PER_TILE, _SAMP), jnp.int32),
            pltpu.VMEM((_PER_TILE, _SAMP), jnp.float32),
            pltpu.SemaphoreType.DMA,
        ],
    )
    def k(masks_hbm, rowid_hbm, flat_hbm, out_hbm, rid_v, rows_v, flat_v, out_v, sem):
        wid = lax.axis_index("s") * 2 + lax.axis_index("c")
        base = wid * _PER_TILE
        pltpu.sync_copy(rowid_hbm.at[pl.ds(base, _PER_TILE)], rid_v)
        pltpu.async_copy(masks_hbm.at[rid_v], rows_v, sem).wait()
        pltpu.sync_copy(flat_hbm.at[pl.ds(base, _PER_TILE)], flat_v)

        def outer(p, _):
            pidx = jnp.full((16,), p, jnp.int32)

            def inner(c, _):
                iv = flat_v[p, pl.ds(c * 16, 16)]
                vals = plsc.load_gather(rows_v, [pidx, iv])
                out_v[p, pl.ds(c * 16, 16)] = vals
                return 0

            lax.fori_loop(0, _SAMP // 16, inner, 0)
            return 0

        lax.fori_loop(0, _PER_TILE, outer, 0)
        pltpu.sync_copy(out_v, out_hbm.at[pl.ds(base, _PER_TILE)])

    return k(masks_t, rowids, flat)


def kernel(proposals, gt_class_ids, gt_boxes, gt_masks):
    pt = jnp.transpose(proposals, (0, 2, 1))
    pt = jnp.pad(pt, ((0, 0), (0, 0), (0, _NPAD - _N)))
    pt = pt.reshape(_B, 4, _ROWS, 128)
    gtt = jnp.transpose(gt_boxes, (0, 2, 1))

    order = pl.pallas_call(
        _sort_kernel,
        grid=(_B,),
        in_specs=[
            pl.BlockSpec(memory_space=pltpu.SMEM),
            pl.BlockSpec((1, 4, _ROWS, 128), lambda b: (b, 0, 0, 0)),
        ],
        out_specs=pl.BlockSpec((1, _SORT_ROWS, 128), lambda b: (b, 0, 0)),
        out_shape=jax.ShapeDtypeStruct((_B, _SORT_ROWS, 128), jnp.int32),
    )(gt_boxes, pt)

    selout = _sc_select(order.reshape(_B, _SORT_N),
                        proposals.reshape(_B, _N * 4),
                        jnp.asarray(_RANKS32))
    rt = jnp.take(
        jnp.transpose(selout.reshape(_B, 8, 4, _SEL_PER_TILE),
                      (0, 2, 1, 3)).reshape(_B, 4, 8 * _SEL_PER_TILE),
        jnp.asarray(_SEL_COLS), axis=2)
    rois = jnp.transpose(rt, (0, 2, 1))
    cls_in = gt_class_ids.reshape(_B, 1, _G)
    lin = jnp.stack([jnp.linspace(0.0, 1.0, _MH), jnp.linspace(0.0, 1.0, _MW)])

    cls, dels, rid, flat = pl.pallas_call(
        _targets_kernel,
        grid=(_B,),
        in_specs=[
            pl.BlockSpec((1, 4, _TOT), lambda b: (b, 0, 0)),
            pl.BlockSpec((1, 4, _G), lambda b: (b, 0, 0)),
            pl.BlockSpec((1, 1, _G), lambda b: (b, 0, 0)),
            pl.BlockSpec((2, _MH), lambda b: (0, 0)),
        ],
        out_specs=[
            pl.BlockSpec((1, 1, _POS), lambda b: (b, 0, 0)),
            pl.BlockSpec((1, 4, _POS), lambda b: (b, 0, 0)),
            pl.BlockSpec((1, 1, _POS), lambda b: (b, 0, 0)),
            pl.BlockSpec((1, _POS, _MH, _MW), lambda b: (b, 0, 0, 0)),
        ],
        out_shape=[
            jax.ShapeDtypeStruct((_B, 1, _POS), jnp.int32),
            jax.ShapeDtypeStruct((_B, 4, _POS), jnp.float32),
            jax.ShapeDtypeStruct((_B, 1, _POS), jnp.int32),
            jax.ShapeDtypeStruct((_B, _POS, _MH, _MW), jnp.int32),
        ],
    )(rt, gtt, cls_in, lin.astype(jnp.float32))

    masks_t = jnp.transpose(gt_masks, (0, 3, 1, 2)).reshape(_B * _G, _H * _W)
    masks_t = jnp.pad(masks_t, ((0, 0), (0, _MASK_LEN - _H * _W)))
    rid_flat = rid.reshape(_B * _POS)
    pad_ids = (np.arange(_SC_ROWS - _B * _POS) * 7) % (_B * _G)
    rid_pad = jnp.concatenate([rid_flat, jnp.asarray(pad_ids, jnp.int32)])
    flat_pad = jnp.pad(flat.reshape(_B * _POS, _SAMP),
                       ((0, _SC_ROWS - _B * _POS), (0, 0)))

    sampled = _sc_mask_gather(masks_t, rid_pad, flat_pad)

    masks_pos = sampled[: _B * _POS].reshape(_B, _POS, _MH, _MW)
    masks_out = jnp.concatenate(
        [masks_pos, jnp.zeros((_B, _NEG, _MH, _MW), jnp.float32)], axis=1)

    cls_out = jnp.concatenate(
        [cls.reshape(_B, _POS), jnp.zeros((_B, _NEG), jnp.int32)], axis=1)
    dels_out = jnp.concatenate(
        [jnp.transpose(dels, (0, 2, 1)), jnp.zeros((_B, _NEG, 4), jnp.float32)],
        axis=1)
    return rois, cls_out, dels_out, masks_out
